# baseline (speedup 1.0000x reference)
import functools

import jax
import jax.numpy as jnp
from jax import lax
from jax.experimental import pallas as pl
from jax.experimental.pallas import tpu as pltpu
from jax.experimental.pallas import tpu_sc as plsc

_LANES = 16
_NUM_TILES = 16
_CHUNK = 128
_Q = 64


def _round_up(v, m):
    return (v + m - 1) // m * m


def _matmul_quarters(x, w):
    n, d_in = x.shape
    d_out = w.shape[1]
    blk = 2000
    grid = n // blk
    nq = d_out // _Q

    def body(x_ref, w_ref, *outs):
        s = jnp.dot(x_ref[...], w_ref[...], preferred_element_type=jnp.float32)
        for q in range(nq):
            outs[q][...] = s[:, q * _Q:(q + 1) * _Q]

    return pl.pallas_call(
        body,
        grid=(grid,),
        in_specs=[
            pl.BlockSpec((blk, d_in), lambda i: (i, 0)),
            pl.BlockSpec((d_in, d_out), lambda i: (0, 0)),
        ],
        out_specs=[pl.BlockSpec((blk, _Q), lambda i: (i, 0))] * nq,
        out_shape=[jax.ShapeDtypeStruct((n, _Q), jnp.float32)] * nq,
    )(x, w)


def _make_sc_scatter(n, n_pad, ept):
    rows_per_tile = n_pad // _NUM_TILES
    chunks = ept // _CHUNK
    assert chunks % 2 == 0
    mesh = plsc.VectorSubcoreMesh(core_axis_name="c", subcore_axis_name="s")

    @functools.partial(
        pl.kernel,
        out_type=[jax.ShapeDtypeStruct((n_pad, _Q), jnp.float32)] * 4,
        mesh=mesh,
        compiler_params=pltpu.CompilerParams(use_tc_tiling_on_sc=False),
        scratch_types=[
            pltpu.VMEM((chunks, _CHUNK), jnp.int32),
            pltpu.VMEM((_CHUNK,), jnp.int32),
            pltpu.VMEM((_CHUNK,), jnp.int32),
            pltpu.VMEM((_CHUNK,), jnp.float32),
            pltpu.VMEM((_CHUNK,), jnp.float32),
            pltpu.VMEM((_CHUNK, _Q), jnp.float32),
            pltpu.VMEM((_CHUNK, _Q), jnp.float32),
            pltpu.VMEM_SHARED((n, _Q), jnp.float32),
            pltpu.VMEM_SHARED((n_pad, _Q), jnp.float32),
            pltpu.SemaphoreType.DMA,
            pltpu.SemaphoreType.DMA,
            pltpu.SemaphoreType.DMA,
            pltpu.SemaphoreType.DMA,
            pltpu.SemaphoreType.DMA,
        ],
    )
    def sc_kernel(supq0, supq1, supq2, supq3, rowh, colh, ewh,
                  bq0, bq1, bq2, bq3, o0, o1, o2, o3,
                  rowv, c0, c1, w0, w1, gbuf0, gbuf1, table, acc,
                  gsem, ssem, csem0, csem1, tsem):
        c = lax.axis_index("c")
        s = lax.axis_index("s")
        r0 = s * rows_per_tile
        dnums = lax.GatherDimensionNumbers(
            offset_dims=(), collapsed_slice_dims=(0,), start_index_map=(0,))

        def run_phase(supq, biasq, outq, stage_rows):
            def gstart(k, buf):
                pltpu.async_copy(table.at[rowv.at[k]], buf, gsem)

            def gwait(k, buf):
                pltpu.make_async_copy(table.at[rowv.at[k]], buf, gsem).wait()

            def sstart(buf, cbuf):
                pltpu.async_copy(buf, acc.at[cbuf], ssem, add=True)

            def swait(buf, cbuf):
                pltpu.make_async_copy(buf, acc.at[cbuf], ssem).wait()

            def cwstart(k, cbuf, wbuf, sem):
                pltpu.async_copy(colh.at[s, k, :], cbuf, sem)
                pltpu.async_copy(ewh.at[s, k, :], wbuf, sem)

            def cwwait(k, cbuf, wbuf, sem):
                pltpu.make_async_copy(colh.at[s, k, :], cbuf, sem).wait()
                pltpu.make_async_copy(ewh.at[s, k, :], wbuf, sem).wait()

            def scale(buf, wbuf):
                @pl.loop(0, _CHUNK // _LANES)
                def _grp(g):
                    wvec = wbuf[pl.ds(g * _LANES, _LANES)]
                    for t in range(_LANES):
                        spl = lax.gather(
                            wvec,
                            jnp.full((_LANES, 1), t, jnp.int32),
                            dnums, (1,),
                            mode=lax.GatherScatterMode.PROMISE_IN_BOUNDS,
                        )
                        e = g * _LANES + t
                        for j in range(_Q // _LANES):
                            sl = pl.ds(j * _LANES, _LANES)
                            buf[e, sl] = buf[e, sl] * spl

            @pl.when(s == 0)
            def _():
                pltpu.async_copy(supq, table, tsem)

            db = pltpu.async_copy(
                biasq.at[pl.ds(r0, rows_per_tile), :],
                acc.at[pl.ds(r0, rows_per_tile)], ssem)
            if stage_rows:
                pltpu.async_copy(rowh.at[s, :, :], rowv, gsem).wait()
            cwstart(0, c0, w0, csem0)
            cwstart(1, c1, w1, csem1)
            db.wait()

            @pl.when(s == 0)
            def _():
                pltpu.make_async_copy(supq, table, tsem).wait()

            plsc.subcore_barrier()
            gstart(0, gbuf0)

            @pl.loop(0, chunks, step=2)
            def _pair(k):
                gwait(k, gbuf0)

                @pl.when(k > 0)
                def _():
                    swait(gbuf1, c1)
                gstart(k + 1, gbuf1)

                @pl.when(k > 0)
                def _():
                    cwstart(k + 1, c1, w1, csem1)
                cwwait(k, c0, w0, csem0)
                scale(gbuf0, w0)
                sstart(gbuf0, c0)

                gwait(k + 1, gbuf1)
                swait(gbuf0, c0)

                @pl.when(k + 2 < chunks)
                def _():
                    gstart(k + 2, gbuf0)
                    cwstart(k + 2, c0, w0, csem0)
                cwwait(k + 1, c1, w1, csem1)
                scale(gbuf1, w1)
                sstart(gbuf1, c1)

            swait(gbuf1, c1)
            plsc.subcore_barrier()
            pltpu.sync_copy(
                acc.at[pl.ds(r0, rows_per_tile)],
                outq.at[pl.ds(r0, rows_per_tile), :],
            )
            plsc.subcore_barrier()

        @pl.when(c == 0)
        def _():
            run_phase(supq0, bq0, o0, True)
            run_phase(supq1, bq1, o1, False)

        @pl.when(c == 1)
        def _():
            run_phase(supq2, bq2, o2, True)
            run_phase(supq3, bq3, o3, False)

    return sc_kernel


def kernel(x, edge_index, edge_weight, W, b):
    n, _ = x.shape
    d_out = W.shape[1]
    e = edge_weight.shape[0]

    supq = _matmul_quarters(x, W)

    n_pad = _round_up(n, _NUM_TILES * 8)
    ept = _round_up(-(-e // _NUM_TILES), 2 * _CHUNK)
    e_pad = ept * _NUM_TILES
    pad = e_pad - e

    row = edge_index[0]
    col = edge_index[1]
    shp = (_NUM_TILES, ept // _CHUNK, _CHUNK)
    rowp = jnp.concatenate([row, jnp.zeros((pad,), jnp.int32)]).reshape(shp)
    colp = jnp.concatenate([col, jnp.zeros((pad,), jnp.int32)]).reshape(shp)
    ewp = jnp.concatenate(
        [edge_weight, jnp.zeros((pad,), jnp.float32)]).reshape(shp)
    biasq = [jnp.broadcast_to(b[q * _Q:(q + 1) * _Q], (n_pad, _Q))
             for q in range(d_out // _Q)]

    sc = _make_sc_scatter(n, n_pad, ept)
    outs = sc(*supq, rowp, colp, ewp, *biasq)
    return jnp.concatenate(outs, axis=1)[:n]


# speedup vs baseline: 1.3145x; 1.3145x over previous
import functools

import jax
import jax.numpy as jnp
from jax import lax
from jax.experimental import pallas as pl
from jax.experimental.pallas import tpu as pltpu
from jax.experimental.pallas import tpu_sc as plsc

_LANES = 16
_NUM_TILES = 16
_CHUNK = 128


def _round_up(v, m):
    return (v + m - 1) // m * m


def _matmul_halves(x, w):
    n, d_in = x.shape
    d_out = w.shape[1]
    h = d_out // 2
    blk = 2000
    grid = n // blk

    def body(x_ref, w_ref, out0_ref, out1_ref):
        s = jnp.dot(x_ref[...], w_ref[...], preferred_element_type=jnp.float32)
        out0_ref[...] = s[:, :h].astype(jnp.bfloat16)
        out1_ref[...] = s[:, h:].astype(jnp.bfloat16)

    return pl.pallas_call(
        body,
        grid=(grid,),
        in_specs=[
            pl.BlockSpec((blk, d_in), lambda i: (i, 0)),
            pl.BlockSpec((d_in, d_out), lambda i: (0, 0)),
        ],
        out_specs=[
            pl.BlockSpec((blk, h), lambda i: (i, 0)),
            pl.BlockSpec((blk, h), lambda i: (i, 0)),
        ],
        out_shape=[
            jax.ShapeDtypeStruct((n, h), jnp.bfloat16),
            jax.ShapeDtypeStruct((n, h), jnp.bfloat16),
        ],
    )(x, w)


def _make_sc_scatter(n_pad, h, ept):
    rows_per_tile = n_pad // _NUM_TILES
    chunks = ept // _CHUNK
    assert chunks % 2 == 0
    mesh = plsc.VectorSubcoreMesh(core_axis_name="c", subcore_axis_name="s")

    @functools.partial(
        pl.kernel,
        out_type=jax.ShapeDtypeStruct((n_pad, 2 * h), jnp.float32),
        mesh=mesh,
        compiler_params=pltpu.CompilerParams(use_tc_tiling_on_sc=False, needs_layout_passes=False),
        scratch_types=[
            pltpu.VMEM((chunks, _CHUNK), jnp.int32),
            pltpu.VMEM((_CHUNK,), jnp.int32),
            pltpu.VMEM((_CHUNK,), jnp.int32),
            pltpu.VMEM((_CHUNK,), jnp.float32),
            pltpu.VMEM((_CHUNK,), jnp.float32),
            pltpu.VMEM((_CHUNK, h), jnp.bfloat16),
            pltpu.VMEM((_CHUNK, h), jnp.bfloat16),
            pltpu.VMEM((_CHUNK, h), jnp.float32),
            pltpu.VMEM_SHARED((n_pad, h), jnp.float32),
            pltpu.SemaphoreType.DMA,
            pltpu.SemaphoreType.DMA,
            pltpu.SemaphoreType.DMA,
            pltpu.SemaphoreType.DMA,
        ],
    )
    def sc_kernel(sup0, sup1, rowh, colh, ewh, biash, out_hbm,
                  rowv, c0, c1, w0, w1, gbuf0, gbuf1, mbuf, acc,
                  gsem, ssem, csem0, csem1):
        c = lax.axis_index("c")
        s = lax.axis_index("s")
        r0 = s * rows_per_tile
        dnums = lax.GatherDimensionNumbers(
            offset_dims=(), collapsed_slice_dims=(0,), start_index_map=(0,))

        def run(sup, coff):
            def gstart(k, buf):
                pltpu.async_copy(sup.at[rowv.at[k]], buf, gsem)

            def gwait(k, buf):
                pltpu.make_async_copy(sup.at[rowv.at[k]], buf, gsem).wait()

            def sstart(buf, cbuf):
                pltpu.async_copy(buf, acc.at[cbuf], ssem, add=True)

            def swait(buf, cbuf):
                pltpu.make_async_copy(buf, acc.at[cbuf], ssem).wait()

            def cwstart(k, cbuf, wbuf, sem):
                pltpu.async_copy(colh.at[s, k, :], cbuf, sem)
                pltpu.async_copy(ewh.at[s, k, :], wbuf, sem)

            def cwwait(k, cbuf, wbuf, sem):
                pltpu.make_async_copy(colh.at[s, k, :], cbuf, sem).wait()
                pltpu.make_async_copy(ewh.at[s, k, :], wbuf, sem).wait()

            def scale(buf, wbuf):
                @pl.loop(0, _CHUNK // _LANES)
                def _grp(g):
                    wvec = wbuf[pl.ds(g * _LANES, _LANES)]
                    for t in range(_LANES):
                        spl = lax.gather(
                            wvec,
                            jnp.full((_LANES, 1), t, jnp.int32),
                            dnums, (1,),
                            mode=lax.GatherScatterMode.PROMISE_IN_BOUNDS,
                        )
                        e = g * _LANES + t
                        for j in range(h // (2 * _LANES)):
                            pkt = buf[e, pl.ds(j * 2 * _LANES, 2 * _LANES)]
                            lo, hi = plsc.unpack(
                                pkt, format=plsc.PackFormat.INTERLEAVED)
                            sl_lo = pl.ds(j * 2 * _LANES, _LANES)
                            sl_hi = pl.ds(j * 2 * _LANES + _LANES, _LANES)
                            mbuf[e, sl_lo] = lo * spl
                            mbuf[e, sl_hi] = hi * spl

            dr = pltpu.async_copy(rowh.at[s, :, :], rowv, gsem)
            db = pltpu.async_copy(
                biash.at[pl.ds(r0, rows_per_tile), pl.ds(coff, h)],
                acc.at[pl.ds(r0, rows_per_tile)], ssem)
            dr.wait()
            cwstart(0, c0, w0, csem0)
            gstart(0, gbuf0)
            cwstart(1, c1, w1, csem1)
            db.wait()
            plsc.subcore_barrier()

            @pl.loop(0, chunks, step=2)
            def _pair(k):
                gwait(k, gbuf0)

                @pl.when(k > 0)
                def _():
                    swait(mbuf, c1)
                gstart(k + 1, gbuf1)

                @pl.when(k > 0)
                def _():
                    cwstart(k + 1, c1, w1, csem1)
                cwwait(k, c0, w0, csem0)
                scale(gbuf0, w0)
                sstart(mbuf, c0)

                gwait(k + 1, gbuf1)
                swait(mbuf, c0)

                @pl.when(k + 2 < chunks)
                def _():
                    gstart(k + 2, gbuf0)
                    cwstart(k + 2, c0, w0, csem0)
                cwwait(k + 1, c1, w1, csem1)
                scale(gbuf1, w1)
                sstart(mbuf, c1)

            swait(mbuf, c1)
            plsc.subcore_barrier()
            pltpu.sync_copy(
                acc.at[pl.ds(r0, rows_per_tile)],
                out_hbm.at[pl.ds(r0, rows_per_tile), pl.ds(coff, h)],
            )

        @pl.when(c == 0)
        def _():
            run(sup0, 0)

        @pl.when(c == 1)
        def _():
            run(sup1, h)

    return sc_kernel


def kernel(x, edge_index, edge_weight, W, b):
    n, _ = x.shape
    d_out = W.shape[1]
    h = d_out // 2
    e = edge_weight.shape[0]

    idx = []
    for k in range(d_out // 32):
        for i in range(16):
            idx.extend((32 * k + i, 32 * k + 16 + i))
    w_perm = W[:, jnp.array(idx, dtype=jnp.int32)]
    sup0, sup1 = _matmul_halves(x, w_perm)

    n_pad = _round_up(n, _NUM_TILES * 8)
    ept = _round_up(-(-e // _NUM_TILES), 2 * _CHUNK)
    e_pad = ept * _NUM_TILES
    pad = e_pad - e

    row = edge_index[0]
    col = edge_index[1]
    shp = (_NUM_TILES, ept // _CHUNK, _CHUNK)
    rowp = jnp.concatenate([row, jnp.zeros((pad,), jnp.int32)]).reshape(shp)
    colp = jnp.concatenate([col, jnp.zeros((pad,), jnp.int32)]).reshape(shp)
    ewp = jnp.concatenate(
        [edge_weight, jnp.zeros((pad,), jnp.float32)]).reshape(shp)
    bias_full = jnp.broadcast_to(b, (n_pad, d_out))

    sc = _make_sc_scatter(n_pad, h, ept)
    out = sc(sup0, sup1, rowp, colp, ewp, bias_full)
    return out[:n]


# speedup vs baseline: 1.4538x; 1.1060x over previous
import functools

import jax
import jax.numpy as jnp
from jax import lax
from jax.experimental import pallas as pl
from jax.experimental.pallas import tpu as pltpu
from jax.experimental.pallas import tpu_sc as plsc

_LANES = 16
_NUM_TILES = 16
_CHUNK = 64
_DEPTH = 4


def _round_up(v, m):
    return (v + m - 1) // m * m


def _matmul_halves(x, w):
    n, d_in = x.shape
    d_out = w.shape[1]
    h = d_out // 2
    blk = 2000
    grid = n // blk

    def body(x_ref, w_ref, out0_ref, out1_ref):
        s = jnp.dot(x_ref[...], w_ref[...], preferred_element_type=jnp.float32)
        out0_ref[...] = s[:, :h].astype(jnp.bfloat16)
        out1_ref[...] = s[:, h:].astype(jnp.bfloat16)

    return pl.pallas_call(
        body,
        grid=(grid,),
        in_specs=[
            pl.BlockSpec((blk, d_in), lambda i: (i, 0)),
            pl.BlockSpec((d_in, d_out), lambda i: (0, 0)),
        ],
        out_specs=[
            pl.BlockSpec((blk, h), lambda i: (i, 0)),
            pl.BlockSpec((blk, h), lambda i: (i, 0)),
        ],
        out_shape=[
            jax.ShapeDtypeStruct((n, h), jnp.bfloat16),
            jax.ShapeDtypeStruct((n, h), jnp.bfloat16),
        ],
    )(x, w)


def _make_sc_scatter(n_pad, h, ept):
    rows_per_tile = n_pad // _NUM_TILES
    chunks = ept // _CHUNK
    assert chunks % _DEPTH == 0 and chunks >= 2 * _DEPTH
    mesh = plsc.VectorSubcoreMesh(core_axis_name="c", subcore_axis_name="s")

    @functools.partial(
        pl.kernel,
        out_type=jax.ShapeDtypeStruct((n_pad, 2 * h), jnp.float32),
        mesh=mesh,
        compiler_params=pltpu.CompilerParams(
            use_tc_tiling_on_sc=False, needs_layout_passes=False),
        scratch_types=[
            pltpu.VMEM((chunks, _CHUNK), jnp.int32),
            [pltpu.VMEM((_CHUNK,), jnp.int32)] * _DEPTH,
            [pltpu.VMEM((_CHUNK,), jnp.float32)] * _DEPTH,
            [pltpu.VMEM((_CHUNK, h), jnp.bfloat16)] * _DEPTH,
            [pltpu.VMEM((_CHUNK, h), jnp.float32)] * 2,
            pltpu.VMEM_SHARED((n_pad, h), jnp.float32),
            [pltpu.SemaphoreType.DMA] * _DEPTH,
            [pltpu.SemaphoreType.DMA] * 2,
            [pltpu.SemaphoreType.DMA] * _DEPTH,
        ],
    )
    def sc_kernel(sup0, sup1, rowh, colh, ewh, biash, out_hbm,
                  rowv, cb, wb, gb, mb, acc, gsem, ssem, csem):
        c = lax.axis_index("c")
        s = lax.axis_index("s")
        r0 = s * rows_per_tile
        dnums = lax.GatherDimensionNumbers(
            offset_dims=(), collapsed_slice_dims=(0,), start_index_map=(0,))

        def run(sup, coff):
            def gstart(k, r):
                pltpu.async_copy(sup.at[rowv.at[k]], gb[r], gsem[r])

            def gwait(k, r):
                pltpu.make_async_copy(sup.at[rowv.at[k]], gb[r], gsem[r]).wait()

            def sstart(m, r):
                pltpu.async_copy(mb[m], acc.at[cb[r]], ssem[m], add=True)

            def swait(m, r):
                pltpu.make_async_copy(mb[m], acc.at[cb[r]], ssem[m]).wait()

            def cwstart(k, r):
                pltpu.async_copy(colh.at[s, k, :], cb[r], csem[r])
                pltpu.async_copy(ewh.at[s, k, :], wb[r], csem[r])

            def cwwait(k, r):
                pltpu.make_async_copy(colh.at[s, k, :], cb[r], csem[r]).wait()
                pltpu.make_async_copy(ewh.at[s, k, :], wb[r], csem[r]).wait()

            def scale(r, m):
                @pl.loop(0, _CHUNK // _LANES)
                def _grp(g):
                    wvec = wb[r][pl.ds(g * _LANES, _LANES)]
                    for t in range(_LANES):
                        spl = lax.gather(
                            wvec,
                            jnp.full((_LANES, 1), t, jnp.int32),
                            dnums, (1,),
                            mode=lax.GatherScatterMode.PROMISE_IN_BOUNDS,
                        )
                        e = g * _LANES + t
                        for j in range(h // (2 * _LANES)):
                            pkt = gb[r][e, pl.ds(j * 2 * _LANES, 2 * _LANES)]
                            lo, hi = plsc.unpack(
                                pkt, format=plsc.PackFormat.INTERLEAVED)
                            sl_lo = pl.ds(j * 2 * _LANES, _LANES)
                            sl_hi = pl.ds(j * 2 * _LANES + _LANES, _LANES)
                            mb[m][e, sl_lo] = lo * spl
                            mb[m][e, sl_hi] = hi * spl

            db = pltpu.async_copy(
                biash.at[pl.ds(r0, rows_per_tile), pl.ds(coff, h)],
                acc.at[pl.ds(r0, rows_per_tile)], ssem[0])
            pltpu.async_copy(rowh.at[s, :, :], rowv, gsem[0]).wait()
            db.wait()
            plsc.subcore_barrier()

            for r in range(_DEPTH):
                gstart(r, r)
            cwstart(0, 0)
            cwstart(1, 1)

            @pl.loop(0, chunks, step=_DEPTH)
            def _ring(k):
                for r in range(_DEPTH):
                    kr = k + r
                    m = r % 2
                    gwait(kr, r)

                    @pl.when(kr >= 2)
                    def _():
                        swait(m, (r + 2) % _DEPTH)

                    @pl.when(kr + 2 < chunks)
                    def _():
                        cwstart(kr + 2, (r + 2) % _DEPTH)
                    cwwait(kr, r)
                    scale(r, m)
                    sstart(m, r)

                    @pl.when(kr + _DEPTH < chunks)
                    def _():
                        gstart(kr + _DEPTH, r)

            swait(chunks % 2, (chunks - 2) % _DEPTH)
            swait((chunks - 1) % 2, (chunks - 1) % _DEPTH)
            plsc.subcore_barrier()
            pltpu.sync_copy(
                acc.at[pl.ds(r0, rows_per_tile)],
                out_hbm.at[pl.ds(r0, rows_per_tile), pl.ds(coff, h)],
            )

        @pl.when(c == 0)
        def _():
            run(sup0, 0)

        @pl.when(c == 1)
        def _():
            run(sup1, h)

    return sc_kernel


def kernel(x, edge_index, edge_weight, W, b):
    n, _ = x.shape
    d_out = W.shape[1]
    h = d_out // 2
    e = edge_weight.shape[0]

    idx = []
    for k in range(d_out // 32):
        for i in range(16):
            idx.extend((32 * k + i, 32 * k + 16 + i))
    w_perm = W[:, jnp.array(idx, dtype=jnp.int32)]
    sup0, sup1 = _matmul_halves(x, w_perm)

    n_pad = _round_up(n, _NUM_TILES * 8)
    ept = _round_up(-(-e // _NUM_TILES), _DEPTH * _CHUNK)
    e_pad = ept * _NUM_TILES
    pad = e_pad - e

    row = edge_index[0]
    col = edge_index[1]
    shp = (_NUM_TILES, ept // _CHUNK, _CHUNK)
    rowp = jnp.concatenate([row, jnp.zeros((pad,), jnp.int32)]).reshape(shp)
    colp = jnp.concatenate([col, jnp.zeros((pad,), jnp.int32)]).reshape(shp)
    ewp = jnp.concatenate(
        [edge_weight, jnp.zeros((pad,), jnp.float32)]).reshape(shp)
    bias_full = jnp.broadcast_to(b, (n_pad, d_out))

    sc = _make_sc_scatter(n_pad, h, ept)
    out = sc(sup0, sup1, rowp, colp, ewp, bias_full)
    return out[:n]
